# baseline (speedup 1.0000x reference)
import functools

import jax
import jax.numpy as jnp
from jax import lax
from jax.experimental import pallas as pl
from jax.experimental.pallas import tpu as pltpu
from jax.experimental.pallas import tpu_sc as plsc


def _prep_body(lab_row_ref, lab_col_ref, idx_row_ref, idx_col_ref,
               keep_ref, minv_ref):
    lab_row = lab_row_ref[...]
    lab_col = lab_col_ref[...]
    idx_row = idx_row_ref[...]
    idx_col = idx_col_ref[...]
    b = lab_row.shape[1]
    eq_lab = lab_col == lab_row
    eq_idx = idx_col == idx_row
    ia = lax.broadcasted_iota(jnp.int32, (b, b), 0)
    ij = lax.broadcasted_iota(jnp.int32, (b, b), 1)
    dup = eq_lab & eq_idx & (ia < ij)
    keep = jnp.where(jnp.any(dup, axis=0, keepdims=True), 0.0, 1.0)
    m = jnp.sum(eq_lab.astype(jnp.float32) * keep, axis=1, keepdims=True)
    keep_ref[...] = keep
    minv_ref[...] = 1.0 / m


def _prep(labels, indices):
    b = labels.shape[0]
    keep, minv = pl.pallas_call(
        _prep_body,
        out_shape=[jax.ShapeDtypeStruct((1, b), jnp.float32),
                   jax.ShapeDtypeStruct((b, 1), jnp.float32)],
    )(labels.reshape(1, b), labels.reshape(b, 1),
      indices.reshape(1, b), indices.reshape(b, 1))
    return keep.reshape(b), minv


def _make_sc_route(b, bt):
    info = plsc.get_sparse_core_info()
    nc, ns, lanes = info.num_cores, info.num_subcores, info.num_lanes
    nw = nc * ns
    nch = b // lanes
    mesh = plsc.VectorSubcoreMesh(core_axis_name="c", subcore_axis_name="s")

    @functools.partial(
        pl.kernel, mesh=mesh,
        out_type=[jax.ShapeDtypeStruct((nw, lanes), jnp.int32),
                  jax.ShapeDtypeStruct((nw, b), jnp.int32),
                  jax.ShapeDtypeStruct((nw, b), jnp.int32)],
        scratch_types=[
            pltpu.VMEM((b,), jnp.int32),
            pltpu.VMEM((b + 16,), jnp.int32),
            pltpu.VMEM((b,), jnp.float32),
            pltpu.VMEM((b + 16,), jnp.int32),
            pltpu.VMEM((b + 16,), jnp.int32),
            pltpu.VMEM((lanes,), jnp.int32),
        ],
    )
    def sc_route(idx_hbm, lab_hbm, keep_hbm, cnt_hbm, cj_hbm, lj_hbm,
                 idx_v, lab_v, keep_v, cj_v, lj_v, cnt_v):
        wid = lax.axis_index("s") * nc + lax.axis_index("c")
        pltpu.sync_copy(idx_hbm, idx_v)
        pltpu.sync_copy(lab_hbm, lab_v.at[pl.ds(0, b)])
        pltpu.sync_copy(keep_hbm, keep_v)

        base = wid * bt
        bl = [jnp.full((lanes,), lab_v[pl.ds(base + t, lanes)][0], jnp.int32)
              for t in range(bt)]

        def chunk_body(c, cur):
            lc = lab_v[pl.ds(c * lanes, lanes)]
            ic = idx_v[pl.ds(c * lanes, lanes)]
            kc = keep_v[pl.ds(c * lanes, lanes)]
            mem = jnp.where(lc == bl[0], 1, 0)
            for t in range(1, bt):
                mem = jnp.maximum(mem, jnp.where(lc == bl[t], 1, 0))
            mski = jnp.where(kc > 0.0, mem, 0)
            for t in range(lanes):
                cj_v[pl.ds(cur, lanes)] = jnp.full((lanes,), ic[t], jnp.int32)
                lj_v[pl.ds(cur, lanes)] = jnp.full((lanes,), lc[t], jnp.int32)
                cur = cur + mski[t]
            return cur

        total = lax.fori_loop(0, nch, chunk_body, jnp.int32(0))
        cnt_v[...] = jnp.full((lanes,), total, jnp.int32)
        pltpu.sync_copy(cnt_v, cnt_hbm.at[wid])
        pltpu.sync_copy(cj_v.at[pl.ds(0, b)], cj_hbm.at[wid])
        pltpu.sync_copy(lj_v.at[pl.ds(0, b)], lj_hbm.at[wid])

    return sc_route


def _main_body(cnt_ref, cj_ref, lj_ref, lab_ref, minv_ref, x_ref,
               out_ref, acc_ref):
    bt = x_ref.shape[0]
    r = pl.program_id(0)
    nblk = pl.num_programs(0)

    @pl.when(r == 0)
    def _():
        acc_ref[0, 0] = 0.0

    x = x_ref[...]
    mx = jnp.max(x, axis=1, keepdims=True)
    ssum = jnp.sum(jnp.exp(x - mx), axis=1, keepdims=True)
    logz = mx + jnp.log(ssum)

    lab_blk = lab_ref[...]
    minv_blk = minv_ref[...]
    col_iota = lax.broadcasted_iota(jnp.int32, (128, 1), 0)
    row_iota = lax.broadcasted_iota(jnp.int32, (bt, 1), 0)
    nlists = cnt_ref.shape[0]
    sub = bt // nlists

    def make_body(q):
        inrows = (row_iota >= q * sub) & (row_iota < (q + 1) * sub)

        def body_k(k, acc):
            c = cj_ref[q, 0, k]
            lab_j = lj_ref[q, 0, k]
            start = pl.multiple_of((c // 128) * 128, 128)
            xt = x_ref[:, pl.ds(start, 128)]
            onehot = jnp.where(col_iota == c % 128, 1.0, 0.0)
            col = jax.lax.dot_general(
                xt, onehot, (((1,), (0,)), ((), ())),
                preferred_element_type=jnp.float32)
            wcol = jnp.where((lab_blk == lab_j) & inrows, minv_blk, 0.0)
            return acc + col * wcol
        return body_k

    acc = jnp.zeros((bt, 1), jnp.float32)
    for q in range(nlists):
        acc = lax.fori_loop(0, cnt_ref[q, 0, 0], make_body(q), acc)
    acc_ref[0, 0] = acc_ref[0, 0] + jnp.sum(logz) - jnp.sum(acc)

    @pl.when(r == nblk - 1)
    def _():
        out_ref[...] = jnp.full((1, 1), acc_ref[0, 0] / (bt * nblk))


def _main(features, counts, cj, lj, labels, minv, bt, lt):
    b, v = features.shape
    nblk = b // bt
    nl = b // lt
    npl = bt // lt
    return pl.pallas_call(
        _main_body,
        grid=(nblk,),
        in_specs=[
            pl.BlockSpec((npl, 1, 16), lambda r: (r, 0, 0),
                         memory_space=pltpu.SMEM),
            pl.BlockSpec((npl, 1, b), lambda r: (r, 0, 0),
                         memory_space=pltpu.SMEM),
            pl.BlockSpec((npl, 1, b), lambda r: (r, 0, 0),
                         memory_space=pltpu.SMEM),
            pl.BlockSpec((bt, 1), lambda r: (r, 0)),
            pl.BlockSpec((bt, 1), lambda r: (r, 0)),
            pl.BlockSpec((bt, v), lambda r: (r, 0)),
        ],
        out_specs=pl.BlockSpec((1, 1), lambda r: (0, 0)),
        out_shape=jax.ShapeDtypeStruct((1, 1), jnp.float32),
        scratch_shapes=[pltpu.SMEM((1, 1), jnp.float32)],
    )(counts.reshape(nl, 1, 16), cj.reshape(nl, 1, b),
      lj.reshape(nl, 1, b), labels.reshape(b, 1), minv, features)


def kernel(features, indices, labels):
    b, v = features.shape
    lt = 32
    bt = 32
    keep, minv = _prep(labels, indices)
    counts, cj, lj = _make_sc_route(b, lt)(indices, labels, keep)
    loss = _main(features, counts, cj, lj, labels, minv, bt, lt)
    return loss.reshape(())


# speedup vs baseline: 1.4581x; 1.4581x over previous
import functools

import jax
import jax.numpy as jnp
from jax import lax
from jax.experimental import pallas as pl
from jax.experimental.pallas import tpu as pltpu
from jax.experimental.pallas import tpu_sc as plsc


def _prep_body(lab_row_ref, lab_col_ref, idx_row_ref, idx_col_ref,
               keep_ref, minv_ref):
    lab_row = lab_row_ref[...]
    lab_col = lab_col_ref[...]
    idx_row = idx_row_ref[...]
    idx_col = idx_col_ref[...]
    b = lab_row.shape[1]
    eq_lab = lab_col == lab_row
    eq_idx = idx_col == idx_row
    ia = lax.broadcasted_iota(jnp.int32, (b, b), 0)
    ij = lax.broadcasted_iota(jnp.int32, (b, b), 1)
    dup = eq_lab & eq_idx & (ia < ij)
    keep = jnp.where(jnp.any(dup, axis=0, keepdims=True), 0.0, 1.0)
    m = jnp.sum(eq_lab.astype(jnp.float32) * keep, axis=1, keepdims=True)
    keep_ref[...] = keep
    minv_ref[...] = 1.0 / m


def _prep(labels, indices):
    b = labels.shape[0]
    keep, minv = pl.pallas_call(
        _prep_body,
        out_shape=[jax.ShapeDtypeStruct((1, b), jnp.float32),
                   jax.ShapeDtypeStruct((b, 1), jnp.float32)],
    )(labels.reshape(1, b), labels.reshape(b, 1),
      indices.reshape(1, b), indices.reshape(b, 1))
    return keep.reshape(b), minv


def _make_sc_route(b, bt):
    info = plsc.get_sparse_core_info()
    nc, ns, lanes = info.num_cores, info.num_subcores, info.num_lanes
    nw = nc * ns
    nch = b // lanes
    mesh = plsc.VectorSubcoreMesh(core_axis_name="c", subcore_axis_name="s")

    @functools.partial(
        pl.kernel, mesh=mesh,
        out_type=[jax.ShapeDtypeStruct((nw, lanes), jnp.int32),
                  jax.ShapeDtypeStruct((nw, b), jnp.int32),
                  jax.ShapeDtypeStruct((nw, b), jnp.int32)],
        scratch_types=[
            pltpu.VMEM((b,), jnp.int32),
            pltpu.VMEM((b + 16,), jnp.int32),
            pltpu.VMEM((b,), jnp.float32),
            pltpu.VMEM((b + 16,), jnp.int32),
            pltpu.VMEM((b + 16,), jnp.int32),
            pltpu.VMEM((lanes,), jnp.int32),
        ],
    )
    def sc_route(idx_hbm, lab_hbm, keep_hbm, cnt_hbm, cj_hbm, lj_hbm,
                 idx_v, lab_v, keep_v, cj_v, lj_v, cnt_v):
        wid = lax.axis_index("s") * nc + lax.axis_index("c")
        pltpu.sync_copy(idx_hbm, idx_v)
        pltpu.sync_copy(lab_hbm, lab_v.at[pl.ds(0, b)])
        pltpu.sync_copy(keep_hbm, keep_v)

        base = wid * bt
        bl = [jnp.full((lanes,), lab_v[pl.ds(base + t, lanes)][0], jnp.int32)
              for t in range(bt)]

        def chunk_body(c, cur):
            lc = lab_v[pl.ds(c * lanes, lanes)]
            ic = idx_v[pl.ds(c * lanes, lanes)]
            kc = keep_v[pl.ds(c * lanes, lanes)]
            mem = jnp.where(lc == bl[0], 1, 0)
            for t in range(1, bt):
                mem = jnp.maximum(mem, jnp.where(lc == bl[t], 1, 0))
            mski = jnp.where(kc > 0.0, mem, 0)
            for t in range(lanes):
                cj_v[pl.ds(cur, lanes)] = jnp.full((lanes,), ic[t], jnp.int32)
                lj_v[pl.ds(cur, lanes)] = jnp.full((lanes,), lc[t], jnp.int32)
                cur = cur + mski[t]
            return cur

        total = lax.fori_loop(0, nch, chunk_body, jnp.int32(0))
        cnt_v[...] = jnp.full((lanes,), total, jnp.int32)
        pltpu.sync_copy(cnt_v, cnt_hbm.at[wid])
        pltpu.sync_copy(cj_v.at[pl.ds(0, b)], cj_hbm.at[wid])
        pltpu.sync_copy(lj_v.at[pl.ds(0, b)], lj_hbm.at[wid])

    return sc_route


def _main_body(cnt_ref, cj_ref, lj_ref, lab_ref, minv_ref, x_ref,
               out_ref, acc_ref):
    bt = x_ref.shape[0]
    r = pl.program_id(0)
    nblk = pl.num_programs(0)

    @pl.when(r == 0)
    def _():
        acc_ref[0, 0] = 0.0

    x = x_ref[...]
    mx = jnp.max(x, axis=1, keepdims=True)
    ssum = jnp.sum(jnp.exp(x - mx), axis=1, keepdims=True)
    logz = mx + jnp.log(ssum)

    lab_blk = lab_ref[...]
    minv_blk = minv_ref[...]
    lane_iota = lax.broadcasted_iota(jnp.int32, (bt, 128), 1)
    row_iota = lax.broadcasted_iota(jnp.int32, (bt, 1), 0)
    nlists = cnt_ref.shape[0]
    sub = bt // nlists

    unroll = 8

    def make_body(q, n):
        inrows = (row_iota >= q * sub) & (row_iota < (q + 1) * sub)

        def body_t(t, acc):
            for i in range(unroll):
                k = t * unroll + i
                kc = jnp.minimum(k, n - 1)
                c = cj_ref[q, 0, kc]
                lab_j = lj_ref[q, 0, kc]
                start = pl.multiple_of((c // 128) * 128, 128)
                xt = x_ref[:, pl.ds(start, 128)]
                wcol = jnp.where((lab_blk == lab_j) & inrows & (k < n),
                                 minv_blk, 0.0)
                acc = acc + jnp.where(lane_iota == c % 128, xt * wcol, 0.0)
            return acc
        return body_t

    acc = jnp.zeros((bt, 128), jnp.float32)
    for q in range(nlists):
        n = cnt_ref[q, 0, 0]
        acc = lax.fori_loop(0, (n + unroll - 1) // unroll,
                            make_body(q, n), acc)
    acc_ref[0, 0] = acc_ref[0, 0] + jnp.sum(logz) - jnp.sum(acc)

    @pl.when(r == nblk - 1)
    def _():
        out_ref[...] = jnp.full((1, 1), acc_ref[0, 0] / (bt * nblk))


def _main(features, counts, cj, lj, labels, minv, bt, lt):
    b, v = features.shape
    nblk = b // bt
    nl = b // lt
    npl = bt // lt
    return pl.pallas_call(
        _main_body,
        grid=(nblk,),
        in_specs=[
            pl.BlockSpec((npl, 1, 16), lambda r: (r, 0, 0),
                         memory_space=pltpu.SMEM),
            pl.BlockSpec((npl, 1, b), lambda r: (r, 0, 0),
                         memory_space=pltpu.SMEM),
            pl.BlockSpec((npl, 1, b), lambda r: (r, 0, 0),
                         memory_space=pltpu.SMEM),
            pl.BlockSpec((bt, 1), lambda r: (r, 0)),
            pl.BlockSpec((bt, 1), lambda r: (r, 0)),
            pl.BlockSpec((bt, v), lambda r: (r, 0)),
        ],
        out_specs=pl.BlockSpec((1, 1), lambda r: (0, 0)),
        out_shape=jax.ShapeDtypeStruct((1, 1), jnp.float32),
        scratch_shapes=[pltpu.SMEM((1, 1), jnp.float32)],
    )(counts.reshape(nl, 1, 16), cj.reshape(nl, 1, b),
      lj.reshape(nl, 1, b), labels.reshape(b, 1), minv, features)


def kernel(features, indices, labels):
    b, v = features.shape
    lt = 32
    bt = 32
    keep, minv = _prep(labels, indices)
    counts, cj, lj = _make_sc_route(b, lt)(indices, labels, keep)
    loss = _main(features, counts, cj, lj, labels, minv, bt, lt)
    return loss.reshape(())


# speedup vs baseline: 1.4727x; 1.0101x over previous
import functools

import jax
import jax.numpy as jnp
from jax import lax
from jax.experimental import pallas as pl
from jax.experimental.pallas import tpu as pltpu
from jax.experimental.pallas import tpu_sc as plsc


def _prep_body(lab_row_ref, lab_col_ref, idx_row_ref, idx_col_ref,
               keep_ref, minv_ref):
    lab_row = lab_row_ref[...]
    lab_col = lab_col_ref[...]
    idx_row = idx_row_ref[...]
    idx_col = idx_col_ref[...]
    b = lab_row.shape[1]
    eq_lab = lab_col == lab_row
    eq_idx = idx_col == idx_row
    ia = lax.broadcasted_iota(jnp.int32, (b, b), 0)
    ij = lax.broadcasted_iota(jnp.int32, (b, b), 1)
    dup = eq_lab & eq_idx & (ia < ij)
    keep = jnp.where(jnp.any(dup, axis=0, keepdims=True), 0.0, 1.0)
    m = jnp.sum(eq_lab.astype(jnp.float32) * keep, axis=1, keepdims=True)
    keep_ref[...] = keep
    minv_ref[...] = 1.0 / m


def _prep(labels, indices):
    b = labels.shape[0]
    keep, minv = pl.pallas_call(
        _prep_body,
        out_shape=[jax.ShapeDtypeStruct((1, b), jnp.float32),
                   jax.ShapeDtypeStruct((b, 1), jnp.float32)],
    )(labels.reshape(1, b), labels.reshape(b, 1),
      indices.reshape(1, b), indices.reshape(b, 1))
    return keep.reshape(b), minv


def _make_sc_route(b, bt):
    info = plsc.get_sparse_core_info()
    nc, ns, lanes = info.num_cores, info.num_subcores, info.num_lanes
    nw = nc * ns
    nch = b // lanes
    mesh = plsc.VectorSubcoreMesh(core_axis_name="c", subcore_axis_name="s")

    @functools.partial(
        pl.kernel, mesh=mesh,
        out_type=[jax.ShapeDtypeStruct((nw, lanes), jnp.int32),
                  jax.ShapeDtypeStruct((nw, b), jnp.int32),
                  jax.ShapeDtypeStruct((nw, b), jnp.int32)],
        scratch_types=[
            pltpu.VMEM((b,), jnp.int32),
            pltpu.VMEM((b + 16,), jnp.int32),
            pltpu.VMEM((b,), jnp.float32),
            pltpu.VMEM((b + 16,), jnp.int32),
            pltpu.VMEM((b + 16,), jnp.int32),
            pltpu.VMEM((lanes,), jnp.int32),
        ],
    )
    def sc_route(idx_hbm, lab_hbm, keep_hbm, cnt_hbm, cj_hbm, lj_hbm,
                 idx_v, lab_v, keep_v, cj_v, lj_v, cnt_v):
        wid = lax.axis_index("s") * nc + lax.axis_index("c")
        pltpu.sync_copy(idx_hbm, idx_v)
        pltpu.sync_copy(lab_hbm, lab_v.at[pl.ds(0, b)])
        pltpu.sync_copy(keep_hbm, keep_v)

        base = wid * bt
        bl = [jnp.full((lanes,), lab_v[pl.ds(base + t, lanes)][0], jnp.int32)
              for t in range(bt)]

        def chunk_body(c, cur):
            lc = lab_v[pl.ds(c * lanes, lanes)]
            ic = idx_v[pl.ds(c * lanes, lanes)]
            kc = keep_v[pl.ds(c * lanes, lanes)]
            mem = jnp.where(lc == bl[0], 1, 0)
            for t in range(1, bt):
                mem = jnp.maximum(mem, jnp.where(lc == bl[t], 1, 0))
            mski = jnp.where(kc > 0.0, mem, 0)
            for t in range(lanes):
                cj_v[pl.ds(cur, lanes)] = jnp.full((lanes,), ic[t], jnp.int32)
                lj_v[pl.ds(cur, lanes)] = jnp.full((lanes,), lc[t], jnp.int32)
                cur = cur + mski[t]
            return cur

        total = lax.fori_loop(0, nch, chunk_body, jnp.int32(0))
        cnt_v[...] = jnp.full((lanes,), total, jnp.int32)
        pltpu.sync_copy(cnt_v, cnt_hbm.at[wid])
        pltpu.sync_copy(cj_v.at[pl.ds(0, b)], cj_hbm.at[wid])
        pltpu.sync_copy(lj_v.at[pl.ds(0, b)], lj_hbm.at[wid])

    return sc_route


def _main_body(cnt_ref, cj_ref, lj_ref, lab_ref, minv_ref, x_ref,
               out_ref, acc_ref):
    bt = x_ref.shape[0]
    r = pl.program_id(0)
    nblk = pl.num_programs(0)

    @pl.when(r == 0)
    def _():
        acc_ref[0, 0] = 0.0

    x = x_ref[...]
    mx = jnp.max(x, axis=1, keepdims=True)
    ssum = jnp.sum(jnp.exp(x - mx), axis=1, keepdims=True)
    logz = mx + jnp.log(ssum)

    lab_blk = lab_ref[...]
    minv_blk = minv_ref[...]
    lane_iota = lax.broadcasted_iota(jnp.int32, (bt, 128), 1)
    row_iota = lax.broadcasted_iota(jnp.int32, (bt, 1), 0)
    nlists = cnt_ref.shape[0]
    sub = bt // nlists

    unroll = 16

    def make_body(q, n):
        inrows = (row_iota >= q * sub) & (row_iota < (q + 1) * sub)

        def body_t(t, acc):
            for i in range(unroll):
                k = t * unroll + i
                kc = jnp.minimum(k, n - 1)
                c = cj_ref[q, 0, kc]
                lab_j = lj_ref[q, 0, kc]
                start = pl.multiple_of((c // 128) * 128, 128)
                xt = x_ref[:, pl.ds(start, 128)]
                wcol = jnp.where((lab_blk == lab_j) & inrows & (k < n),
                                 minv_blk, 0.0)
                acc = acc + jnp.where(lane_iota == c % 128, xt * wcol, 0.0)
            return acc
        return body_t

    acc = jnp.zeros((bt, 128), jnp.float32)
    for q in range(nlists):
        n = cnt_ref[q, 0, 0]
        acc = lax.fori_loop(0, (n + unroll - 1) // unroll,
                            make_body(q, n), acc)
    acc_ref[0, 0] = acc_ref[0, 0] + jnp.sum(logz) - jnp.sum(acc)

    @pl.when(r == nblk - 1)
    def _():
        out_ref[...] = jnp.full((1, 1), acc_ref[0, 0] / (bt * nblk))


def _main(features, counts, cj, lj, labels, minv, bt, lt):
    b, v = features.shape
    nblk = b // bt
    nl = b // lt
    npl = bt // lt
    return pl.pallas_call(
        _main_body,
        grid=(nblk,),
        in_specs=[
            pl.BlockSpec((npl, 1, 16), lambda r: (r, 0, 0),
                         memory_space=pltpu.SMEM),
            pl.BlockSpec((npl, 1, b), lambda r: (r, 0, 0),
                         memory_space=pltpu.SMEM),
            pl.BlockSpec((npl, 1, b), lambda r: (r, 0, 0),
                         memory_space=pltpu.SMEM),
            pl.BlockSpec((bt, 1), lambda r: (r, 0)),
            pl.BlockSpec((bt, 1), lambda r: (r, 0)),
            pl.BlockSpec((bt, v), lambda r: (r, 0)),
        ],
        out_specs=pl.BlockSpec((1, 1), lambda r: (0, 0)),
        out_shape=jax.ShapeDtypeStruct((1, 1), jnp.float32),
        scratch_shapes=[pltpu.SMEM((1, 1), jnp.float32)],
    )(counts.reshape(nl, 1, 16), cj.reshape(nl, 1, b),
      lj.reshape(nl, 1, b), labels.reshape(b, 1), minv, features)


def kernel(features, indices, labels):
    b, v = features.shape
    lt = 32
    bt = 32
    keep, minv = _prep(labels, indices)
    counts, cj, lj = _make_sc_route(b, lt)(indices, labels, keep)
    loss = _main(features, counts, cj, lj, labels, minv, bt, lt)
    return loss.reshape(())
